# baseline (speedup 1.0000x reference)
import jax
import jax.numpy as jnp
from jax import lax
from jax.experimental import pallas as pl
from jax.experimental.pallas import tpu as pltpu
from jax.experimental.pallas import tpu_sc as plsc

N = 10000
E = 320000
D = 128
NCLS = 16
NG = 64

NC = 2
NS = 16
NW = NC * NS
EPT = E // NW
CH = 80
NCH = EPT // CH
N_PAD = 10240
RPT = N_PAD // NS
DDEG = 16

DH = D // NC
EPT2 = E // NS
NCH2 = EPT2 // CH
NP2 = 10112
RPT2 = NP2 // NS

_mesh = plsc.VectorSubcoreMesh(core_axis_name="c", subcore_axis_name="s")
_sc_params = pltpu.CompilerParams(use_tc_tiling_on_sc=False)


def _sc_degree(dst_t):
    @pl.kernel(
        out_type=jax.ShapeDtypeStruct((NC, N_PAD, DDEG), jnp.float32),
        mesh=_mesh,
        compiler_params=_sc_params,
        scratch_types=[
            pltpu.VMEM((NCH, CH), jnp.int32),
            pltpu.VMEM((CH, DDEG), jnp.float32),
            pltpu.VMEM_SHARED((N_PAD, DDEG), jnp.float32),
            pltpu.SemaphoreType.DMA,
        ],
    )
    def k(dst_hbm, out_hbm, dst_v, ones_v, acc_sh, sem):
        cid = lax.axis_index("c")
        sid = lax.axis_index("s")
        wid = cid * NS + sid
        pltpu.sync_copy(dst_hbm.at[wid], dst_v)

        zero16 = jnp.zeros((16,), jnp.float32)

        @pl.loop(0, CH)
        def _(r):
            ones_v[r, :] = zero16

        @pl.loop(0, RPT // CH)
        def _(i):
            pltpu.sync_copy(ones_v, acc_sh.at[pl.ds(sid * RPT + i * CH, CH)])

        one16 = jnp.full((16,), 1.0, jnp.float32)

        @pl.loop(0, CH)
        def _(r):
            ones_v[r, :] = one16

        plsc.subcore_barrier()

        @pl.loop(0, NCH)
        def _(j):
            pltpu.sync_copy(ones_v, acc_sh.at[dst_v.at[j]], add=True)

        plsc.subcore_barrier()
        pltpu.sync_copy(acc_sh.at[pl.ds(sid * RPT, RPT)],
                        out_hbm.at[cid, pl.ds(sid * RPT, RPT)])

    return k(dst_t)


def _sc_aggregate(hs, src_t, dst_t):
    @pl.kernel(
        out_type=jax.ShapeDtypeStruct((NP2, D), jnp.float32),
        mesh=_mesh,
        compiler_params=_sc_params,
        scratch_types=[
            pltpu.VMEM((NCH2, CH), jnp.int32),
            pltpu.VMEM((NCH2, CH), jnp.int32),
            pltpu.VMEM((CH, DH), jnp.float32),
            pltpu.VMEM((CH, DH), jnp.float32),
            pltpu.VMEM_SHARED((N, DH), jnp.float32),
            pltpu.VMEM_SHARED((NP2, DH), jnp.float32),
            pltpu.SemaphoreType.DMA,
            pltpu.SemaphoreType.DMA,
        ],
    )
    def k(hs_hbm, src_hbm, dst_hbm, out_hbm,
          src_v, dst_v, rows_a, rows_b, hs_sh, acc_sh, sem_a, sem_b):
        cid = lax.axis_index("c")
        sid = lax.axis_index("s")
        pltpu.sync_copy(src_hbm.at[sid], src_v)
        pltpu.sync_copy(dst_hbm.at[sid], dst_v)
        nrt = N // NS
        pltpu.sync_copy(
            hs_hbm.at[pl.ds(sid * nrt, nrt), pl.ds(cid * DH, DH)],
            hs_sh.at[pl.ds(sid * nrt, nrt)])

        zero16 = jnp.zeros((16,), jnp.float32)

        @pl.loop(0, CH)
        def _(r):
            @pl.loop(0, DH // 16)
            def _(c):
                rows_a[r, pl.ds(c * 16, 16)] = zero16

        @pl.loop(0, RPT2 // 79)
        def _(i):
            pltpu.sync_copy(rows_a.at[pl.ds(0, 79)],
                            acc_sh.at[pl.ds(sid * RPT2 + i * 79, 79)])

        plsc.subcore_barrier()

        def gather(j, buf, sem):
            pltpu.async_copy(hs_sh.at[src_v.at[j]], buf, sem)

        def wait(j, buf, sem):
            pltpu.make_async_copy(hs_sh.at[src_v.at[j]], buf, sem).wait()

        def scat(j, buf):
            pltpu.sync_copy(buf, acc_sh.at[dst_v.at[j]], add=True)

        gather(0, rows_a, sem_a)

        @pl.loop(0, NCH2 - 2, step=2)
        def _(j):
            gather(j + 1, rows_b, sem_b)
            wait(j, rows_a, sem_a)
            scat(j, rows_a)
            gather(j + 2, rows_a, sem_a)
            wait(j + 1, rows_b, sem_b)
            scat(j + 1, rows_b)

        gather(NCH2 - 1, rows_b, sem_b)
        wait(NCH2 - 2, rows_a, sem_a)
        scat(NCH2 - 2, rows_a)
        wait(NCH2 - 1, rows_b, sem_b)
        scat(NCH2 - 1, rows_b)

        plsc.subcore_barrier()
        pltpu.sync_copy(acc_sh.at[pl.ds(sid * RPT2, RPT2)],
                        out_hbm.at[pl.ds(sid * RPT2, RPT2),
                                   pl.ds(cid * DH, DH)])

    return k(hs, src_t, dst_t)


def _tc_matmul(x, W):
    def body(x_ref, w_ref, o_ref):
        o_ref[...] = jnp.dot(x_ref[...], w_ref[...],
                             preferred_element_type=jnp.float32)
    return pl.pallas_call(
        body, out_shape=jax.ShapeDtypeStruct((x.shape[0], W.shape[1]),
                                             jnp.float32))(x, W)


def _tc_scale(h, degp):
    def body(h_ref, degp_ref, hs_ref, dinv_ref):
        deg = degp_ref[0, :N, 0:1] + degp_ref[1, :N, 0:1]
        dinv = lax.rsqrt(1.0 + deg)
        dinv_ref[...] = dinv
        hs_ref[...] = h_ref[...] * dinv
    return pl.pallas_call(
        body,
        out_shape=(jax.ShapeDtypeStruct((N, D), jnp.float32),
                   jax.ShapeDtypeStruct((N, 1), jnp.float32)))(h, degp)


def _tc_layer_out(p, hs, dinv, b, W):
    def body(p_ref, hs_ref, dinv_ref, b_ref, w_ref, o_ref):
        agg = p_ref[:N, :] + hs_ref[...]
        out = jnp.maximum(dinv_ref[...] * agg + b_ref[...], 0.0)
        o_ref[...] = dinv_ref[...] * jnp.dot(out, w_ref[...],
                                             preferred_element_type=jnp.float32)
    return pl.pallas_call(
        body, out_shape=jax.ShapeDtypeStruct((N, D), jnp.float32))(
            p, hs, dinv, b, W)


def _tc_head(q, hs, dinv, b, batch2d, Wlin, blin):
    def body(q_ref, hs_ref, dinv_ref, b_ref, batch_ref,
             wl_ref, bl_ref, o_ref):
        agg = q_ref[:N, :] + hs_ref[...]
        out = jnp.maximum(dinv_ref[...] * agg + b_ref[...], 0.0)
        gids = lax.broadcasted_iota(jnp.int32, (NG, N), 0)
        P = (gids == batch_ref[...]).astype(jnp.float32)
        sums = jnp.dot(P, out, preferred_element_type=jnp.float32)
        counts = jnp.sum(P, axis=1, keepdims=True)
        pooled = sums / jnp.maximum(counts, 1.0)
        o_ref[...] = jnp.dot(pooled, wl_ref[...],
                             preferred_element_type=jnp.float32) + bl_ref[...]
    return pl.pallas_call(
        body, out_shape=jax.ShapeDtypeStruct((NG, NCLS), jnp.float32))(
            q, hs, dinv, b, batch2d, Wlin, blin)


@jax.jit
def kernel(x, edge_index, batch, W1, b1, W2, b2, Wlin, blin):
    dst_deg = edge_index[1].reshape(NW, NCH, CH)
    src_t = edge_index[0].reshape(NS, NCH2, CH)
    dst_t = edge_index[1].reshape(NS, NCH2, CH)
    batch2d = batch.reshape(1, N)
    b1r = b1.reshape(1, D)
    b2r = b2.reshape(1, D)

    degp = _sc_degree(dst_deg)
    h1 = _tc_matmul(x, W1)
    hs1, dinv = _tc_scale(h1, degp)

    p = _sc_aggregate(hs1, src_t, dst_t)
    hs2 = _tc_layer_out(p, hs1, dinv, b1r, W2)

    q = _sc_aggregate(hs2, src_t, dst_t)
    return _tc_head(q, hs2, dinv, b2r, batch2d, Wlin, blin)


# speedup vs baseline: 1.3065x; 1.3065x over previous
import jax
import jax.numpy as jnp
from jax import lax
from jax.experimental import pallas as pl
from jax.experimental.pallas import tpu as pltpu
from jax.experimental.pallas import tpu_sc as plsc

N = 10000
E = 320000
D = 128
NCLS = 16
NG = 64

NC = 2
NS = 16
NW = NC * NS
EPT = E // NW
CH = 80
NCH = EPT // CH
N_PAD = 10000
RPT = N_PAD // NS
DDEG = 16

_mesh = plsc.VectorSubcoreMesh(core_axis_name="c", subcore_axis_name="s")
_sc_params = pltpu.CompilerParams(use_tc_tiling_on_sc=False)


def _sc_degree(dst_t):
    @pl.kernel(
        out_type=jax.ShapeDtypeStruct((NC, N_PAD, DDEG), jnp.float32),
        mesh=_mesh,
        compiler_params=_sc_params,
        scratch_types=[
            pltpu.VMEM((NCH, CH), jnp.int32),
            pltpu.VMEM((CH, DDEG), jnp.float32),
            pltpu.VMEM_SHARED((N_PAD, DDEG), jnp.float32),
            pltpu.SemaphoreType.DMA,
        ],
    )
    def k(dst_hbm, out_hbm, dst_v, ones_v, acc_sh, sem):
        cid = lax.axis_index("c")
        sid = lax.axis_index("s")
        wid = cid * NS + sid
        pltpu.sync_copy(dst_hbm.at[wid], dst_v)

        zero16 = jnp.zeros((16,), jnp.float32)

        @pl.loop(0, CH)
        def _(r):
            ones_v[r, :] = zero16

        @pl.loop(0, 7)
        def _(i):
            pltpu.sync_copy(ones_v, acc_sh.at[pl.ds(sid * RPT + i * CH, CH)])
        pltpu.sync_copy(ones_v.at[pl.ds(0, 65)],
                        acc_sh.at[pl.ds(sid * RPT + 560, 65)])

        one16 = jnp.full((16,), 1.0, jnp.float32)

        @pl.loop(0, CH)
        def _(r):
            ones_v[r, :] = one16

        plsc.subcore_barrier()

        @pl.loop(0, NCH)
        def _(j):
            pltpu.sync_copy(ones_v, acc_sh.at[dst_v.at[j]], add=True)

        plsc.subcore_barrier()
        pltpu.sync_copy(acc_sh.at[pl.ds(sid * RPT, RPT)],
                        out_hbm.at[cid, pl.ds(sid * RPT, RPT)])

    return k(dst_t)


def _sc_aggregate(hs, src_t, dst_t):
    @pl.kernel(
        out_type=jax.ShapeDtypeStruct((NC, N_PAD, D), jnp.float32),
        mesh=_mesh,
        compiler_params=_sc_params,
        scratch_types=[
            pltpu.VMEM((NCH, CH), jnp.int32),
            pltpu.VMEM((NCH, CH), jnp.int32),
            pltpu.VMEM((CH, D), jnp.float32),
            pltpu.VMEM((CH, D), jnp.float32),
            pltpu.VMEM((CH, D), jnp.float32),
            pltpu.VMEM_SHARED((N_PAD, D), jnp.float32),
            pltpu.SemaphoreType.DMA,
            pltpu.SemaphoreType.DMA,
            pltpu.SemaphoreType.DMA,
        ],
    )
    def k(hs_hbm, src_hbm, dst_hbm, out_hbm,
          src_v, dst_v, rows_a, rows_b, rows_c, acc_sh,
          sem_a, sem_b, sem_c):
        cid = lax.axis_index("c")
        sid = lax.axis_index("s")
        wid = cid * NS + sid
        pltpu.sync_copy(src_hbm.at[wid], src_v)
        pltpu.sync_copy(dst_hbm.at[wid], dst_v)

        zero16 = jnp.zeros((16,), jnp.float32)

        @pl.loop(0, CH)
        def _(r):
            @pl.loop(0, D // 16)
            def _(c):
                rows_a[r, pl.ds(c * 16, 16)] = zero16

        @pl.loop(0, 7)
        def _(i):
            pltpu.sync_copy(rows_a, acc_sh.at[pl.ds(sid * RPT + i * CH, CH)])
        pltpu.sync_copy(rows_a.at[pl.ds(0, 65)],
                        acc_sh.at[pl.ds(sid * RPT + 560, 65)])

        plsc.subcore_barrier()

        def gather(j, buf, sem):
            pltpu.async_copy(hs_hbm.at[src_v.at[j]], buf, sem)

        def wait(j, buf, sem):
            pltpu.make_async_copy(hs_hbm.at[src_v.at[j]], buf, sem).wait()

        def scat(j, buf):
            pltpu.sync_copy(buf, acc_sh.at[dst_v.at[j]], add=True)

        bufs = ((rows_a, sem_a), (rows_b, sem_b), (rows_c, sem_c))
        gather(0, rows_a, sem_a)
        gather(1, rows_b, sem_b)

        @pl.loop(0, NCH - 2, step=3)
        def _(j):
            for t in range(3):
                buf, sem = bufs[t]
                nbuf, nsem = bufs[(t + 2) % 3]
                wait(j + t, buf, sem)
                gather(j + t + 2, nbuf, nsem)
                scat(j + t, buf)

        wait(NCH - 2, bufs[0][0], bufs[0][1])
        scat(NCH - 2, bufs[0][0])
        wait(NCH - 1, bufs[1][0], bufs[1][1])
        scat(NCH - 1, bufs[1][0])

        plsc.subcore_barrier()
        pltpu.sync_copy(acc_sh.at[pl.ds(sid * RPT, RPT)],
                        out_hbm.at[cid, pl.ds(sid * RPT, RPT)])

    return k(hs, src_t, dst_t)


def _tc_matmul(x, W):
    def body(x_ref, w_ref, o_ref):
        o_ref[...] = jnp.dot(x_ref[...], w_ref[...],
                             preferred_element_type=jnp.float32)
    return pl.pallas_call(
        body, out_shape=jax.ShapeDtypeStruct((x.shape[0], W.shape[1]),
                                             jnp.float32))(x, W)


def _tc_scale(h, degp):
    def body(h_ref, degp_ref, hs_ref, dinv_ref):
        deg = degp_ref[0, :N, 0:1] + degp_ref[1, :N, 0:1]
        dinv = lax.rsqrt(1.0 + deg)
        dinv_ref[...] = dinv
        hs_ref[...] = h_ref[...] * dinv
    return pl.pallas_call(
        body,
        out_shape=(jax.ShapeDtypeStruct((N, D), jnp.float32),
                   jax.ShapeDtypeStruct((N, 1), jnp.float32)))(h, degp)


def _tc_layer_out(p, hs, dinv, b, W):
    def body(p_ref, hs_ref, dinv_ref, b_ref, w_ref, o_ref):
        agg = p_ref[0, :N, :] + p_ref[1, :N, :] + hs_ref[...]
        out = jnp.maximum(dinv_ref[...] * agg + b_ref[...], 0.0)
        o_ref[...] = dinv_ref[...] * jnp.dot(out, w_ref[...],
                                             preferred_element_type=jnp.float32)
    return pl.pallas_call(
        body, out_shape=jax.ShapeDtypeStruct((N, D), jnp.float32))(
            p, hs, dinv, b, W)


def _tc_head(q, hs, dinv, b, batch2d, Wlin, blin):
    def body(q_ref, hs_ref, dinv_ref, b_ref, batch_ref,
             wl_ref, bl_ref, o_ref):
        agg = q_ref[0, :N, :] + q_ref[1, :N, :] + hs_ref[...]
        out = jnp.maximum(dinv_ref[...] * agg + b_ref[...], 0.0)
        gids = lax.broadcasted_iota(jnp.int32, (NG, N), 0)
        P = (gids == batch_ref[...]).astype(jnp.float32)
        sums = jnp.dot(P, out, preferred_element_type=jnp.float32)
        counts = jnp.sum(P, axis=1, keepdims=True)
        pooled = sums / jnp.maximum(counts, 1.0)
        o_ref[...] = jnp.dot(pooled, wl_ref[...],
                             preferred_element_type=jnp.float32) + bl_ref[...]
    return pl.pallas_call(
        body, out_shape=jax.ShapeDtypeStruct((NG, NCLS), jnp.float32))(
            q, hs, dinv, b, batch2d, Wlin, blin)


@jax.jit
def kernel(x, edge_index, batch, W1, b1, W2, b2, Wlin, blin):
    src_t = edge_index[0].reshape(NW, NCH, CH)
    dst_t = edge_index[1].reshape(NW, NCH, CH)
    batch2d = batch.reshape(1, N)
    b1r = b1.reshape(1, D)
    b2r = b2.reshape(1, D)

    degp = _sc_degree(dst_t)
    h1 = _tc_matmul(x, W1)
    hs1, dinv = _tc_scale(h1, degp)

    p = _sc_aggregate(hs1, src_t, dst_t)
    hs2 = _tc_layer_out(p, hs1, dinv, b1r, W2)

    q = _sc_aggregate(hs2, src_t, dst_t)
    return _tc_head(q, hs2, dinv, b2r, batch2d, Wlin, blin)


# speedup vs baseline: 1.3227x; 1.0123x over previous
import jax
import jax.numpy as jnp
from jax import lax
from jax.experimental import pallas as pl
from jax.experimental.pallas import tpu as pltpu
from jax.experimental.pallas import tpu_sc as plsc

N = 10000
E = 320000
D = 128
NCLS = 16
NG = 64

NC = 2
NS = 16
NW = NC * NS
EPT = E // NW
CH = 80
NCH = EPT // CH
N_PAD = 10000
RPT = N_PAD // NS
DDEG = 16

_mesh = plsc.VectorSubcoreMesh(core_axis_name="c", subcore_axis_name="s")
_sc_params = pltpu.CompilerParams(use_tc_tiling_on_sc=False)


CHD = 400
NCHD = EPT // CHD


def _sc_degree(dst_t):
    @pl.kernel(
        out_type=jax.ShapeDtypeStruct((NC, N_PAD, DDEG), jnp.float32),
        mesh=_mesh,
        compiler_params=_sc_params,
        scratch_types=[
            pltpu.VMEM((NCHD, CHD), jnp.int32),
            pltpu.VMEM((CHD, DDEG), jnp.float32),
            pltpu.VMEM_SHARED((N_PAD, DDEG), jnp.float32),
            pltpu.SemaphoreType.DMA,
        ],
    )
    def k(dst_hbm, out_hbm, dst_v, ones_v, acc_sh, sem):
        cid = lax.axis_index("c")
        sid = lax.axis_index("s")
        wid = cid * NS + sid
        pltpu.sync_copy(dst_hbm.at[wid], dst_v)

        zero16 = jnp.zeros((16,), jnp.float32)

        @pl.loop(0, CHD)
        def _(r):
            ones_v[r, :] = zero16

        pltpu.sync_copy(ones_v, acc_sh.at[pl.ds(sid * RPT, CHD)])
        pltpu.sync_copy(ones_v.at[pl.ds(0, 225)],
                        acc_sh.at[pl.ds(sid * RPT + 400, 225)])

        one16 = jnp.full((16,), 1.0, jnp.float32)

        @pl.loop(0, CHD)
        def _(r):
            ones_v[r, :] = one16

        plsc.subcore_barrier()

        @pl.loop(0, NCHD)
        def _(j):
            pltpu.sync_copy(ones_v, acc_sh.at[dst_v.at[j]], add=True)

        plsc.subcore_barrier()
        pltpu.sync_copy(acc_sh.at[pl.ds(sid * RPT, RPT)],
                        out_hbm.at[cid, pl.ds(sid * RPT, RPT)])

    return k(dst_t)


def _sc_aggregate(hs, src_t, dst_t):
    @pl.kernel(
        out_type=jax.ShapeDtypeStruct((NC, N_PAD, D), jnp.float32),
        mesh=_mesh,
        compiler_params=_sc_params,
        scratch_types=[
            pltpu.VMEM((NCH, CH), jnp.int32),
            pltpu.VMEM((NCH, CH), jnp.int32),
            pltpu.VMEM((CH, D), jnp.float32),
            pltpu.VMEM((CH, D), jnp.float32),
            pltpu.VMEM((CH, D), jnp.float32),
            pltpu.VMEM_SHARED((N_PAD, D), jnp.float32),
            pltpu.SemaphoreType.DMA,
            pltpu.SemaphoreType.DMA,
            pltpu.SemaphoreType.DMA,
        ],
    )
    def k(hs_hbm, src_hbm, dst_hbm, out_hbm,
          src_v, dst_v, rows_a, rows_b, rows_c, acc_sh,
          sem_a, sem_b, sem_c):
        cid = lax.axis_index("c")
        sid = lax.axis_index("s")
        wid = cid * NS + sid
        pltpu.sync_copy(src_hbm.at[wid], src_v)
        pltpu.sync_copy(dst_hbm.at[wid], dst_v)

        zero16 = jnp.zeros((16,), jnp.float32)

        @pl.loop(0, CH)
        def _(r):
            @pl.loop(0, D // 16)
            def _(c):
                rows_a[r, pl.ds(c * 16, 16)] = zero16

        @pl.loop(0, 7)
        def _(i):
            pltpu.sync_copy(rows_a, acc_sh.at[pl.ds(sid * RPT + i * CH, CH)])
        pltpu.sync_copy(rows_a.at[pl.ds(0, 65)],
                        acc_sh.at[pl.ds(sid * RPT + 560, 65)])

        plsc.subcore_barrier()

        def gather(j, buf, sem):
            pltpu.async_copy(hs_hbm.at[src_v.at[j]], buf, sem)

        def wait(j, buf, sem):
            pltpu.make_async_copy(hs_hbm.at[src_v.at[j]], buf, sem).wait()

        def scat(j, buf):
            pltpu.sync_copy(buf, acc_sh.at[dst_v.at[j]], add=True)

        bufs = ((rows_a, sem_a), (rows_b, sem_b), (rows_c, sem_c))
        gather(0, rows_a, sem_a)
        gather(1, rows_b, sem_b)

        @pl.loop(0, NCH - 2, step=3)
        def _(j):
            for t in range(3):
                buf, sem = bufs[t]
                nbuf, nsem = bufs[(t + 2) % 3]
                wait(j + t, buf, sem)
                gather(j + t + 2, nbuf, nsem)
                scat(j + t, buf)

        wait(NCH - 2, bufs[0][0], bufs[0][1])
        scat(NCH - 2, bufs[0][0])
        wait(NCH - 1, bufs[1][0], bufs[1][1])
        scat(NCH - 1, bufs[1][0])

        plsc.subcore_barrier()
        pltpu.sync_copy(acc_sh.at[pl.ds(sid * RPT, RPT)],
                        out_hbm.at[cid, pl.ds(sid * RPT, RPT)])

    return k(hs, src_t, dst_t)


def _tc_matmul(x, W):
    def body(x_ref, w_ref, o_ref):
        o_ref[...] = jnp.dot(x_ref[...], w_ref[...],
                             preferred_element_type=jnp.float32)
    return pl.pallas_call(
        body, out_shape=jax.ShapeDtypeStruct((x.shape[0], W.shape[1]),
                                             jnp.float32))(x, W)


def _tc_scale(x, W, degp):
    def body(x_ref, w_ref, degp_ref, hs_ref, dinv_ref):
        deg = degp_ref[0, :N, 0:1] + degp_ref[1, :N, 0:1]
        dinv = lax.rsqrt(1.0 + deg)
        dinv_ref[...] = dinv
        h = jnp.dot(x_ref[...], w_ref[...], preferred_element_type=jnp.float32)
        hs_ref[...] = h * dinv
    return pl.pallas_call(
        body,
        out_shape=(jax.ShapeDtypeStruct((N, D), jnp.float32),
                   jax.ShapeDtypeStruct((N, 1), jnp.float32)))(x, W, degp)


def _tc_layer_out(p, hs, dinv, b, W):
    def body(p_ref, hs_ref, dinv_ref, b_ref, w_ref, o_ref):
        agg = p_ref[0, :N, :] + p_ref[1, :N, :] + hs_ref[...]
        out = jnp.maximum(dinv_ref[...] * agg + b_ref[...], 0.0)
        o_ref[...] = dinv_ref[...] * jnp.dot(out, w_ref[...],
                                             preferred_element_type=jnp.float32)
    return pl.pallas_call(
        body, out_shape=jax.ShapeDtypeStruct((N, D), jnp.float32))(
            p, hs, dinv, b, W)


def _tc_head(q, hs, dinv, b, batch2d, Wlin, blin):
    def body(q_ref, hs_ref, dinv_ref, b_ref, batch_ref,
             wl_ref, bl_ref, o_ref):
        agg = q_ref[0, :N, :] + q_ref[1, :N, :] + hs_ref[...]
        out = jnp.maximum(dinv_ref[...] * agg + b_ref[...], 0.0)
        gids = lax.broadcasted_iota(jnp.int32, (NG, N), 0)
        P = (gids == batch_ref[...]).astype(jnp.float32)
        sums = jnp.dot(P, out, preferred_element_type=jnp.float32)
        counts = jnp.sum(P, axis=1, keepdims=True)
        pooled = sums / jnp.maximum(counts, 1.0)
        o_ref[...] = jnp.dot(pooled, wl_ref[...],
                             preferred_element_type=jnp.float32) + bl_ref[...]
    return pl.pallas_call(
        body, out_shape=jax.ShapeDtypeStruct((NG, NCLS), jnp.float32))(
            q, hs, dinv, b, batch2d, Wlin, blin)


@jax.jit
def kernel(x, edge_index, batch, W1, b1, W2, b2, Wlin, blin):
    src_t = edge_index[0].reshape(NW, NCH, CH)
    dst_t = edge_index[1].reshape(NW, NCH, CH)
    dst_deg = edge_index[1].reshape(NW, NCHD, CHD)
    batch2d = batch.reshape(1, N)
    b1r = b1.reshape(1, D)
    b2r = b2.reshape(1, D)

    degp = _sc_degree(dst_deg)
    hs1, dinv = _tc_scale(x, W1, degp)

    p = _sc_aggregate(hs1, src_t, dst_t)
    hs2 = _tc_layer_out(p, hs1, dinv, b1r, W2)

    q = _sc_aggregate(hs2, src_t, dst_t)
    return _tc_head(q, hs2, dinv, b2r, batch2d, Wlin, blin)
